# Initial kernel scaffold; baseline (speedup 1.0000x reference)
import jax
import jax.numpy as jnp
from jax.experimental import pallas as pl


def kernel(x, edge_index, W, b):
    raise NotImplementedError("write your pallas kernel here")


# speedup vs baseline: 32.8966x; 32.8966x over previous
import functools

import jax
import jax.numpy as jnp
from jax import lax
from jax.experimental import pallas as pl
from jax.experimental.pallas import tpu as pltpu
from jax.experimental.pallas import tpu_sc as plsc

N = 10000
E = 320000
D = 128
KW = 4

NC = 2
NS = 16
NW = NC * NS
L = 16

EPT = E // NW
CH = 80
NCHUNK = EPT // CH
RA = 624
TAIL = N - NS * RA
ZROWS = 104

_mesh = plsc.VectorSubcoreMesh(core_axis_name="c", subcore_axis_name="s")
_sc_params = pltpu.CompilerParams(needs_layout_passes=False)


@functools.partial(
    pl.kernel,
    mesh=_mesh,
    out_type=jax.ShapeDtypeStruct((NW, 1, N), jnp.float32),
    scratch_types=[
        pltpu.VMEM((1, N), jnp.float32),
        pltpu.VMEM((CH,), jnp.int32),
    ],
    compiler_params=_sc_params,
)
def _deg_kernel(dst_hbm, out_hbm, hist_v, idx_v):
    c = lax.axis_index("c")
    s = lax.axis_index("s")
    wid = s * NC + c

    zeros16 = jnp.zeros((L,), jnp.float32)
    ones16 = jnp.ones((L,), jnp.float32)

    def zero_body(i, _):
        hist_v[0, pl.ds(i * L, L)] = zeros16
        return 0

    lax.fori_loop(0, N // L, zero_body, 0)

    def chunk_body(ch, _):
        off = wid * EPT + ch * CH
        pltpu.sync_copy(dst_hbm.at[pl.ds(off, CH)], idx_v)

        def lane_body(j, _):
            idx = idx_v[pl.ds(j * L, L)]
            plsc.addupdate_scatter(hist_v.at[0], [idx], ones16)
            return 0

        lax.fori_loop(0, CH // L, lane_body, 0)
        return 0

    lax.fori_loop(0, NCHUNK, chunk_body, 0)
    pltpu.sync_copy(hist_v, out_hbm.at[wid])


def _prep_body(hists_ref, x_ref, y_ref, dinv_ref):
    deg = jnp.sum(hists_ref[...], axis=0) + 1.0
    dinv = lax.rsqrt(deg)
    y_ref[...] = x_ref[...] * dinv[:, None]
    dinv_ref[...] = dinv[:, None]


_prep = pl.pallas_call(
    _prep_body,
    out_shape=[
        jax.ShapeDtypeStruct((N, D), jnp.float32),
        jax.ShapeDtypeStruct((N, 1), jnp.float32),
    ],
)


@functools.partial(
    pl.kernel,
    mesh=_mesh,
    out_type=jax.ShapeDtypeStruct((NC, N, D), jnp.float32),
    scratch_types=[
        pltpu.VMEM((CH,), jnp.int32),
        pltpu.VMEM((CH,), jnp.int32),
        pltpu.VMEM((CH, D), jnp.float32),
        pltpu.VMEM((ZROWS, D), jnp.float32),
        pltpu.VMEM_SHARED((N, D), jnp.float32),
        pltpu.SemaphoreType.DMA,
    ],
    compiler_params=_sc_params,
)
def _agg_kernel(y_hbm, src_hbm, dst_hbm, out_hbm, src_v, dst_v, rows_v, zbuf_v,
                accum, sem):
    c = lax.axis_index("c")
    s = lax.axis_index("s")
    wid = s * NC + c
    base = s * RA

    @pl.when(c == 0)
    def _():
        pltpu.sync_copy(y_hbm.at[pl.ds(base, RA)], accum.at[pl.ds(base, RA)])

        @pl.when(s == 0)
        def _():
            pltpu.sync_copy(y_hbm.at[pl.ds(NS * RA, TAIL)],
                            accum.at[pl.ds(NS * RA, TAIL)])

    @pl.when(c != 0)
    def _():
        zeros16 = jnp.zeros((L,), jnp.float32)

        def zfill(t, _):
            zbuf_v[t // (D // L), pl.ds((t % (D // L)) * L, L)] = zeros16
            return 0

        lax.fori_loop(0, ZROWS * (D // L), zfill, 0)

        def zcopy(r, _):
            pltpu.sync_copy(zbuf_v,
                            accum.at[pl.ds(base + r * ZROWS, ZROWS)])
            return 0

        lax.fori_loop(0, RA // ZROWS, zcopy, 0)

        @pl.when(s == 0)
        def _():
            pltpu.sync_copy(zbuf_v.at[pl.ds(0, TAIL)],
                            accum.at[pl.ds(NS * RA, TAIL)])

    plsc.subcore_barrier()

    def chunk_body(ch, _):
        off = wid * EPT + ch * CH
        pltpu.sync_copy(src_hbm.at[pl.ds(off, CH)], src_v)
        pltpu.sync_copy(dst_hbm.at[pl.ds(off, CH)], dst_v)
        pltpu.async_copy(y_hbm.at[src_v], rows_v, sem).wait()
        pltpu.sync_copy(rows_v, accum.at[dst_v], add=True)
        return 0

    lax.fori_loop(0, NCHUNK, chunk_body, 0)
    plsc.subcore_barrier()

    pltpu.sync_copy(accum.at[pl.ds(base, RA)],
                    out_hbm.at[c, pl.ds(base, RA)])

    @pl.when(s == 0)
    def _():
        pltpu.sync_copy(accum.at[pl.ds(NS * RA, TAIL)],
                        out_hbm.at[c, pl.ds(NS * RA, TAIL)])


def _out_body(a_ref, dinv_ref, w_ref, b_ref, o_ref):
    t = (a_ref[0] + a_ref[1]) * dinv_ref[...]
    acc = None
    for k in range(KW):
        f = jnp.dot(t, w_ref[k], preferred_element_type=jnp.float32)
        f = jnp.maximum(f + b_ref[k][None, :], 0.0)
        acc = f if acc is None else acc + f
    o_ref[...] = acc


_BN = 400

_out = pl.pallas_call(
    _out_body,
    grid=(N // _BN,),
    in_specs=[
        pl.BlockSpec((NC, _BN, D), lambda i: (0, i, 0)),
        pl.BlockSpec((_BN, 1), lambda i: (i, 0)),
        pl.BlockSpec((KW, D, D), lambda i: (0, 0, 0)),
        pl.BlockSpec((KW, D), lambda i: (0, 0)),
    ],
    out_specs=pl.BlockSpec((_BN, D), lambda i: (i, 0)),
    out_shape=jax.ShapeDtypeStruct((N, D), jnp.float32),
)


@jax.jit
def kernel(x, edge_index, W, b):
    src = edge_index[0]
    dst = edge_index[1]
    hists = _deg_kernel(dst).reshape(NW, N)
    y, dinv = _prep(hists, x)
    partials = _agg_kernel(y, src, dst)
    return _out(partials, dinv, W, b)


# speedup vs baseline: 84.9538x; 2.5824x over previous
import functools

import jax
import jax.numpy as jnp
from jax import lax
from jax.experimental import pallas as pl
from jax.experimental.pallas import tpu as pltpu
from jax.experimental.pallas import tpu_sc as plsc

N = 10000
E = 320000
D = 128
KW = 4

NC = 2
NS = 16
NW = NC * NS
L = 16

EPT = E // NW
CH = 80
NCHUNK = EPT // CH
NBUF = 3
RA = 624
TAIL = N - NS * RA

_mesh = plsc.VectorSubcoreMesh(core_axis_name="c", subcore_axis_name="s")
_sc_params = pltpu.CompilerParams(needs_layout_passes=False)


@functools.partial(
    pl.kernel,
    mesh=_mesh,
    out_type=jax.ShapeDtypeStruct((NW, 1, N), jnp.float32),
    scratch_types=[
        pltpu.VMEM((NCHUNK, CH), jnp.int32),
        pltpu.VMEM((1, N), jnp.float32),
        pltpu.SemaphoreType.DMA,
    ],
    compiler_params=_sc_params,
)
def _deg_kernel(dst3_hbm, out_hbm, dsts_v, hist_v, sem):
    c = lax.axis_index("c")
    s = lax.axis_index("s")
    wid = s * NC + c

    cp = pltpu.async_copy(dst3_hbm.at[wid], dsts_v, sem)

    zeros16 = jnp.zeros((L,), jnp.float32)
    ones16 = jnp.ones((L,), jnp.float32)

    def zero_body(i, _):
        hist_v[0, pl.ds(i * L, L)] = zeros16
        return 0

    lax.fori_loop(0, N // L, zero_body, 0)
    cp.wait()

    def chunk_body(i, _):
        for j in range(CH // L):
            idx = dsts_v[i, pl.ds(j * L, L)]
            plsc.addupdate_scatter(hist_v.at[0], [idx], ones16)
        return 0

    lax.fori_loop(0, NCHUNK, chunk_body, 0)
    pltpu.sync_copy(hist_v, out_hbm.at[wid])


def _prep_body(hists_ref, x_ref, y_ref, dinv_ref):
    deg = jnp.sum(hists_ref[...], axis=0) + 1.0
    dinv = lax.rsqrt(deg)
    y_ref[...] = x_ref[...] * dinv[:, None]
    dinv_ref[...] = dinv[:, None]


_prep = pl.pallas_call(
    _prep_body,
    out_shape=[
        jax.ShapeDtypeStruct((N, D), jnp.float32),
        jax.ShapeDtypeStruct((N, 1), jnp.float32),
    ],
)


@functools.partial(
    pl.kernel,
    mesh=_mesh,
    out_type=jax.ShapeDtypeStruct((NC, N, D), jnp.float32),
    scratch_types=[
        pltpu.VMEM((EPT,), jnp.int32),
        [pltpu.VMEM((1, CH), jnp.int32) for _ in range(NBUF)],
        [pltpu.VMEM((CH, D), jnp.float32) for _ in range(NBUF)],
        pltpu.VMEM_SHARED((N, D), jnp.float32),
        pltpu.SemaphoreType.DMA,
        [pltpu.SemaphoreType.DMA for _ in range(NBUF)],
        [pltpu.SemaphoreType.DMA for _ in range(NBUF)],
    ],
    compiler_params=_sc_params,
)
def _agg_kernel(y_hbm, src_hbm, dst4_hbm, out_hbm, srcs_v, dbufs, rows,
                accum, si0, sds, sgs):
    c = lax.axis_index("c")
    s = lax.axis_index("s")
    wid = s * NC + c
    base = s * RA

    cp_src = pltpu.async_copy(src_hbm.at[pl.ds(wid * EPT, EPT)], srcs_v, si0)

    @pl.when(c == 0)
    def _():
        pltpu.sync_copy(y_hbm.at[pl.ds(base, RA)], accum.at[pl.ds(base, RA)])

        @pl.when(s == 0)
        def _():
            pltpu.sync_copy(y_hbm.at[pl.ds(NS * RA, TAIL)],
                            accum.at[pl.ds(NS * RA, TAIL)])

    @pl.when(c != 0)
    def _():
        zeros16 = jnp.zeros((L,), jnp.float32)

        def zfill(t, _):
            rows[0][t // (D // L), pl.ds((t % (D // L)) * L, L)] = zeros16
            return 0

        lax.fori_loop(0, CH * (D // L), zfill, 0)

        def zcopy(r, _):
            pltpu.sync_copy(rows[0],
                            accum.at[pl.ds(base + r * CH, CH)])
            return 0

        lax.fori_loop(0, RA // CH, zcopy, 0)
        pltpu.sync_copy(rows[0].at[pl.ds(0, RA - (RA // CH) * CH)],
                        accum.at[pl.ds(base + (RA // CH) * CH,
                                       RA - (RA // CH) * CH)])

        @pl.when(s == 0)
        def _():
            pltpu.sync_copy(rows[0].at[pl.ds(0, TAIL)],
                            accum.at[pl.ds(NS * RA, TAIL)])

    cp_src.wait()

    def dst_dma(i, j):
        pltpu.async_copy(dst4_hbm.at[wid, i], dbufs[j], sds[j])

    def dst_wait(j):
        pltpu.make_async_copy(dst4_hbm.at[0, 0], dbufs[j], sds[j]).wait()

    def gather(i, j):
        pltpu.async_copy(y_hbm.at[srcs_v.at[pl.ds(i * CH, CH)]], rows[j],
                         sgs[j])

    def gather_wait(j):
        pltpu.make_async_copy(y_hbm.at[pl.ds(0, CH)], rows[j], sgs[j]).wait()

    def scatter(j):
        pltpu.sync_copy(rows[j], accum.at[dbufs[j].at[0]], add=True)

    for j in range(NBUF):
        dst_dma(j, j)
        gather(j, j)

    plsc.subcore_barrier()

    NQ = (NCHUNK - 1) // NBUF

    def round_body(p, _):
        for j in range(NBUF):
            i = p * NBUF + j
            gather_wait(j)
            dst_wait(j)
            scatter(j)

            @pl.when(i + NBUF < NCHUNK)
            def _():
                dst_dma(i + NBUF, j)
                gather(i + NBUF, j)

        return 0

    lax.fori_loop(0, NQ, round_body, 0)

    for t in range(NQ * NBUF, NCHUNK):
        j = t % NBUF
        gather_wait(j)
        dst_wait(j)
        scatter(j)

    plsc.subcore_barrier()

    pltpu.sync_copy(accum.at[pl.ds(base, RA)],
                    out_hbm.at[c, pl.ds(base, RA)])

    @pl.when(s == 0)
    def _():
        pltpu.sync_copy(accum.at[pl.ds(NS * RA, TAIL)],
                        out_hbm.at[c, pl.ds(NS * RA, TAIL)])


def _out_body(a_ref, dinv_ref, w_ref, b_ref, o_ref):
    t = (a_ref[0] + a_ref[1]) * dinv_ref[...]
    acc = None
    for k in range(KW):
        f = jnp.dot(t, w_ref[k], preferred_element_type=jnp.float32)
        f = jnp.maximum(f + b_ref[k][None, :], 0.0)
        acc = f if acc is None else acc + f
    o_ref[...] = acc


_BN = 400

_out = pl.pallas_call(
    _out_body,
    grid=(N // _BN,),
    in_specs=[
        pl.BlockSpec((NC, _BN, D), lambda i: (0, i, 0)),
        pl.BlockSpec((_BN, 1), lambda i: (i, 0)),
        pl.BlockSpec((KW, D, D), lambda i: (0, 0, 0)),
        pl.BlockSpec((KW, D), lambda i: (0, 0)),
    ],
    out_specs=pl.BlockSpec((_BN, D), lambda i: (i, 0)),
    out_shape=jax.ShapeDtypeStruct((N, D), jnp.float32),
)


@jax.jit
def kernel(x, edge_index, W, b):
    src = edge_index[0]
    dst = edge_index[1]
    dst3 = dst.reshape(NW, NCHUNK, CH)
    dst4 = dst.reshape(NW, NCHUNK, 1, CH)
    hists = _deg_kernel(dst3).reshape(NW, N)
    y, dinv = _prep(hists, x)
    partials = _agg_kernel(y, src, dst4)
    return _out(partials, dinv, W, b)
